# baseline (speedup 1.0000x reference)
import functools

import jax
import jax.numpy as jnp
from jax import lax
from jax.experimental import pallas as pl
from jax.experimental.pallas import tpu as pltpu
from jax.experimental.pallas import tpu_sc as plsc

N = 10000
NP = 10240
E = 320000
D = 128
GAMMA_COEF = 0.5

NC = 2
NS = 16
NW = NC * NS
EP = E // NW
BATCH = 100
NB = EP // BATCH
DEPTH = 2
RP = NP // NS

_mesh = plsc.VectorSubcoreMesh(core_axis_name="c", subcore_axis_name="s")
_sc_params = pltpu.CompilerParams(use_tc_tiling_on_sc=False)


@functools.partial(
    pl.kernel,
    out_type=jax.ShapeDtypeStruct((NC, NP, 16), jnp.float32),
    mesh=_mesh,
    compiler_params=_sc_params,
    scratch_types=[
        pltpu.VMEM((NB, BATCH), jnp.int32),
        pltpu.VMEM((RP, 16), jnp.float32),
        pltpu.VMEM_SHARED((NP, 16), jnp.float32),
        pltpu.SemaphoreType.DMA,
    ],
)
def _deg_kernel(dst_hbm, out_hbm, idx_v, ones_v, deg_sh, sem):
    c = lax.axis_index("c")
    s = lax.axis_index("s")
    wid = c * NS + s

    pltpu.sync_copy(dst_hbm.at[wid], idx_v)

    def fill(i, _):
        ones_v[i] = jnp.full((16,), 1.0, jnp.float32)
        return ()

    lax.fori_loop(0, RP, fill, ())
    pltpu.sync_copy(ones_v, deg_sh.at[pl.ds(s * RP, RP)])
    plsc.subcore_barrier()

    def body(j, _):
        pltpu.sync_copy(ones_v.at[pl.ds(0, BATCH)], deg_sh.at[idx_v.at[j]],
                        add=True)
        return ()

    lax.fori_loop(0, NB, body, ())
    plsc.subcore_barrier()
    pltpu.sync_copy(deg_sh.at[pl.ds(s * RP, RP)],
                    out_hbm.at[c, pl.ds(s * RP, RP)])


@functools.partial(
    pl.kernel,
    out_type=jax.ShapeDtypeStruct((NC, NP, D), jnp.float32),
    mesh=_mesh,
    compiler_params=_sc_params,
    scratch_types=(
        [
            pltpu.VMEM((NB, BATCH), jnp.int32),
            pltpu.VMEM((NB, BATCH), jnp.int32),
        ]
        + [pltpu.VMEM((BATCH, D), jnp.float32) for _ in range(DEPTH)]
        + [pltpu.VMEM_SHARED((NP, D), jnp.float32)]
        + [pltpu.SemaphoreType.DMA for _ in range(DEPTH)]
    ),
)
def _prop_kernel(y_hbm, src_hbm, dst_hbm, out_hbm, si_v, di_v, *scratch):
    rows_bufs = scratch[:DEPTH]
    p_sh = scratch[DEPTH]
    sems = scratch[DEPTH + 1:]
    c = lax.axis_index("c")
    s = lax.axis_index("s")
    wid = c * NS + s

    pltpu.sync_copy(src_hbm.at[wid], si_v)
    pltpu.sync_copy(dst_hbm.at[wid], di_v)
    pltpu.sync_copy(y_hbm.at[pl.ds(s * RP, RP)], p_sh.at[pl.ds(s * RP, RP)])
    plsc.subcore_barrier()

    def start(j, buf, sem):
        pltpu.make_async_copy(y_hbm.at[si_v.at[j]], buf, sem).start()

    def drain_scatter(j, buf, sem):
        pltpu.make_async_copy(y_hbm.at[si_v.at[j]], buf, sem).wait()
        pltpu.sync_copy(buf, p_sh.at[di_v.at[j]], add=True)

    bufs = tuple((rows_bufs[t], sems[t]) for t in range(DEPTH))
    for t in range(DEPTH):
        start(t, *bufs[t])

    def body(i, _):
        j = DEPTH * i
        for t in range(DEPTH):
            jj = j + t

            @pl.when(jj < NB)
            def _(jj=jj, t=t):
                drain_scatter(jj, *bufs[t])

            @pl.when(jj + DEPTH < NB)
            def _(jj=jj, t=t):
                start(jj + DEPTH, *bufs[t])

        return ()

    lax.fori_loop(0, (NB + DEPTH - 1) // DEPTH, body, ())
    plsc.subcore_barrier()
    pltpu.sync_copy(p_sh.at[pl.ds(s * RP, RP)],
                    out_hbm.at[c, pl.ds(s * RP, RP)])


def _mm1_body(x_ref, w_ref, xl_ref):
    xl_ref[...] = jnp.dot(x_ref[...], w_ref[...],
                          preferred_element_type=jnp.float32)


_mm1_call = pl.pallas_call(
    _mm1_body,
    out_shape=jax.ShapeDtypeStruct((N, D), jnp.float32),
)


def _scale1_body(deg_ref, xl_ref, dinv_ref, y_ref):
    deg = deg_ref[0][0:N, 0:1] + deg_ref[1][0:N, 0:1] - 1.0
    dinv = lax.rsqrt(deg)
    dinv_ref[...] = dinv
    y_ref[0:N, :] = dinv * xl_ref[...]
    y_ref[N:NP, :] = jnp.zeros((NP - N, D), jnp.float32)


_scale1_call = pl.pallas_call(
    _scale1_body,
    out_shape=[
        jax.ShapeDtypeStruct((N, 1), jnp.float32),
        jax.ShapeDtypeStruct((NP, D), jnp.float32),
    ],
)


def _out_of_layer(xl_ref, y_ref, p_ref, dinv_ref, b_ref, coef, relu):
    y = y_ref[0:N, :]
    agg = dinv_ref[...] * (p_ref[0][0:N, :] + p_ref[1][0:N, :] - y)
    if coef == 1.0:
        out = agg + b_ref[...]
    else:
        out = (1.0 - coef) * xl_ref[...] + coef * agg + b_ref[...]
    if relu:
        out = jnp.maximum(out, 0.0)
    return out


def _mid_body(xl_ref, y_ref, p_ref, dinv_ref, b_ref, w_ref, *refs,
              coef, relu, keep_out):
    out = _out_of_layer(xl_ref, y_ref, p_ref, dinv_ref, b_ref, coef, relu)
    if keep_out:
        out_ref, xl2_ref, y2_ref = refs
        out_ref[...] = out
    else:
        xl2_ref, y2_ref = refs
    xl2 = jnp.dot(out, w_ref[...], preferred_element_type=jnp.float32)
    xl2_ref[...] = xl2
    y2_ref[0:N, :] = dinv_ref[...] * xl2
    y2_ref[N:NP, :] = jnp.zeros((NP - N, D), jnp.float32)


def _mid_call(coef, relu, keep_out):
    shapes = [
        jax.ShapeDtypeStruct((N, D), jnp.float32),
        jax.ShapeDtypeStruct((NP, D), jnp.float32),
    ]
    if keep_out:
        shapes = [jax.ShapeDtypeStruct((N, D), jnp.float32)] + shapes
    return pl.pallas_call(
        functools.partial(_mid_body, coef=coef, relu=relu, keep_out=keep_out),
        out_shape=shapes,
    )


def _last_body(xl_ref, y_ref, p_ref, dinv_ref, b_ref, out_ref, *, coef, relu):
    out_ref[...] = _out_of_layer(xl_ref, y_ref, p_ref, dinv_ref, b_ref,
                                 coef, relu)


def _last_call(coef, relu):
    return pl.pallas_call(
        functools.partial(_last_body, coef=coef, relu=relu),
        out_shape=jax.ShapeDtypeStruct((N, D), jnp.float32),
    )


def kernel(x, We1, be1, We2, be2, Wd1, bd1, Wd2, bd2, edge_index):
    src3 = edge_index[0].astype(jnp.int32).reshape(NW, NB, BATCH)
    dst3 = edge_index[1].astype(jnp.int32).reshape(NW, NB, BATCH)

    deg2 = _deg_kernel(dst3)
    xl1 = _mm1_call(x, We1)
    dinv, y1 = _scale1_call(deg2, xl1)
    p1 = _prop_kernel(y1, src3, dst3)
    xl2, y2 = _mid_call(1.0, True, False)(
        xl1, y1, p1, dinv, be1.reshape(1, D), We2)
    p2 = _prop_kernel(y2, src3, dst3)
    h, xl3, y3 = _mid_call(1.0, False, True)(
        xl2, y2, p2, dinv, be2.reshape(1, D), Wd1)
    p3 = _prop_kernel(y3, src3, dst3)
    xl4, y4 = _mid_call(GAMMA_COEF, True, False)(
        xl3, y3, p3, dinv, bd1.reshape(1, D), Wd2)
    p4 = _prop_kernel(y4, src3, dst3)
    r = _last_call(GAMMA_COEF, False)(xl4, y4, p4, dinv, bd2.reshape(1, D))
    return (h, r)


# speedup vs baseline: 1.1249x; 1.1249x over previous
import functools

import jax
import jax.numpy as jnp
from jax import lax
from jax.experimental import pallas as pl
from jax.experimental.pallas import tpu as pltpu
from jax.experimental.pallas import tpu_sc as plsc

N = 10000
NP = 10240
E = 320000
D = 128
GAMMA_COEF = 0.5

NC = 2
NS = 16
NW = NC * NS
EP = E // NW
BATCH = 40
NB = EP // BATCH
DEPTH = 5
RP = NP // NS

_mesh = plsc.VectorSubcoreMesh(core_axis_name="c", subcore_axis_name="s")
_sc_params = pltpu.CompilerParams(use_tc_tiling_on_sc=False)


@functools.partial(
    pl.kernel,
    out_type=jax.ShapeDtypeStruct((NC, NP, 16), jnp.float32),
    mesh=_mesh,
    compiler_params=_sc_params,
    scratch_types=[
        pltpu.VMEM((NB, BATCH), jnp.int32),
        pltpu.VMEM((RP, 16), jnp.float32),
        pltpu.VMEM_SHARED((NP, 16), jnp.float32),
        pltpu.SemaphoreType.DMA,
    ],
)
def _deg_kernel(dst_hbm, out_hbm, idx_v, ones_v, deg_sh, sem):
    c = lax.axis_index("c")
    s = lax.axis_index("s")
    wid = c * NS + s

    pltpu.sync_copy(dst_hbm.at[wid], idx_v)

    def fill(i, _):
        ones_v[i] = jnp.full((16,), 1.0, jnp.float32)
        return ()

    lax.fori_loop(0, RP, fill, ())
    pltpu.sync_copy(ones_v, deg_sh.at[pl.ds(s * RP, RP)])
    plsc.subcore_barrier()

    def body(j, _):
        pltpu.sync_copy(ones_v.at[pl.ds(0, BATCH)], deg_sh.at[idx_v.at[j]],
                        add=True)
        return ()

    lax.fori_loop(0, NB, body, ())
    plsc.subcore_barrier()
    pltpu.sync_copy(deg_sh.at[pl.ds(s * RP, RP)],
                    out_hbm.at[c, pl.ds(s * RP, RP)])


@functools.partial(
    pl.kernel,
    out_type=jax.ShapeDtypeStruct((NC, NP, D), jnp.float32),
    mesh=_mesh,
    compiler_params=_sc_params,
    scratch_types=(
        [
            pltpu.VMEM((NB, BATCH), jnp.int32),
            pltpu.VMEM((NB, BATCH), jnp.int32),
        ]
        + [pltpu.VMEM((BATCH, D), jnp.float32) for _ in range(DEPTH)]
        + [pltpu.VMEM_SHARED((NP, D), jnp.float32)]
        + [pltpu.SemaphoreType.DMA for _ in range(DEPTH)]
    ),
)
def _prop_kernel(y_hbm, src_hbm, dst_hbm, out_hbm, si_v, di_v, *scratch):
    rows_bufs = scratch[:DEPTH]
    p_sh = scratch[DEPTH]
    sems = scratch[DEPTH + 1:]
    c = lax.axis_index("c")
    s = lax.axis_index("s")
    wid = c * NS + s

    pltpu.sync_copy(src_hbm.at[wid], si_v)
    pltpu.sync_copy(dst_hbm.at[wid], di_v)
    pltpu.sync_copy(y_hbm.at[pl.ds(s * RP, RP)], p_sh.at[pl.ds(s * RP, RP)])
    plsc.subcore_barrier()

    def start(j, buf, sem):
        pltpu.make_async_copy(y_hbm.at[si_v.at[j]], buf, sem).start()

    def drain_scatter(j, buf, sem):
        pltpu.make_async_copy(y_hbm.at[si_v.at[j]], buf, sem).wait()
        pltpu.sync_copy(buf, p_sh.at[di_v.at[j]], add=True)

    bufs = tuple((rows_bufs[t], sems[t]) for t in range(DEPTH))
    for t in range(DEPTH):
        start(t, *bufs[t])

    def body(i, _):
        j = DEPTH * i
        for t in range(DEPTH):
            jj = j + t

            @pl.when(jj < NB)
            def _(jj=jj, t=t):
                drain_scatter(jj, *bufs[t])

            @pl.when(jj + DEPTH < NB)
            def _(jj=jj, t=t):
                start(jj + DEPTH, *bufs[t])

        return ()

    lax.fori_loop(0, (NB + DEPTH - 1) // DEPTH, body, ())
    plsc.subcore_barrier()
    pltpu.sync_copy(p_sh.at[pl.ds(s * RP, RP)],
                    out_hbm.at[c, pl.ds(s * RP, RP)])


def _mm1_body(x_ref, w_ref, xl_ref):
    xl_ref[...] = jnp.dot(x_ref[...], w_ref[...],
                          preferred_element_type=jnp.float32)


_mm1_call = pl.pallas_call(
    _mm1_body,
    out_shape=jax.ShapeDtypeStruct((N, D), jnp.float32),
)


def _scale1_body(deg_ref, xl_ref, dinv_ref, y_ref):
    deg = deg_ref[0][0:N, 0:1] + deg_ref[1][0:N, 0:1] - 1.0
    dinv = lax.rsqrt(deg)
    dinv_ref[...] = dinv
    y_ref[0:N, :] = dinv * xl_ref[...]
    y_ref[N:NP, :] = jnp.zeros((NP - N, D), jnp.float32)


_scale1_call = pl.pallas_call(
    _scale1_body,
    out_shape=[
        jax.ShapeDtypeStruct((N, 1), jnp.float32),
        jax.ShapeDtypeStruct((NP, D), jnp.float32),
    ],
)


def _out_of_layer(xl_ref, y_ref, p_ref, dinv_ref, b_ref, coef, relu):
    y = y_ref[0:N, :]
    agg = dinv_ref[...] * (p_ref[0][0:N, :] + p_ref[1][0:N, :] - y)
    if coef == 1.0:
        out = agg + b_ref[...]
    else:
        out = (1.0 - coef) * xl_ref[...] + coef * agg + b_ref[...]
    if relu:
        out = jnp.maximum(out, 0.0)
    return out


def _mid_body(xl_ref, y_ref, p_ref, dinv_ref, b_ref, w_ref, *refs,
              coef, relu, keep_out):
    out = _out_of_layer(xl_ref, y_ref, p_ref, dinv_ref, b_ref, coef, relu)
    if keep_out:
        out_ref, xl2_ref, y2_ref = refs
        out_ref[...] = out
    else:
        xl2_ref, y2_ref = refs
    xl2 = jnp.dot(out, w_ref[...], preferred_element_type=jnp.float32)
    xl2_ref[...] = xl2
    y2_ref[0:N, :] = dinv_ref[...] * xl2
    y2_ref[N:NP, :] = jnp.zeros((NP - N, D), jnp.float32)


def _mid_call(coef, relu, keep_out):
    shapes = [
        jax.ShapeDtypeStruct((N, D), jnp.float32),
        jax.ShapeDtypeStruct((NP, D), jnp.float32),
    ]
    if keep_out:
        shapes = [jax.ShapeDtypeStruct((N, D), jnp.float32)] + shapes
    return pl.pallas_call(
        functools.partial(_mid_body, coef=coef, relu=relu, keep_out=keep_out),
        out_shape=shapes,
    )


def _last_body(xl_ref, y_ref, p_ref, dinv_ref, b_ref, out_ref, *, coef, relu):
    out_ref[...] = _out_of_layer(xl_ref, y_ref, p_ref, dinv_ref, b_ref,
                                 coef, relu)


def _last_call(coef, relu):
    return pl.pallas_call(
        functools.partial(_last_body, coef=coef, relu=relu),
        out_shape=jax.ShapeDtypeStruct((N, D), jnp.float32),
    )


def kernel(x, We1, be1, We2, be2, Wd1, bd1, Wd2, bd2, edge_index):
    src3 = edge_index[0].astype(jnp.int32).reshape(NW, NB, BATCH)
    dst3 = edge_index[1].astype(jnp.int32).reshape(NW, NB, BATCH)

    deg2 = _deg_kernel(dst3)
    xl1 = _mm1_call(x, We1)
    dinv, y1 = _scale1_call(deg2, xl1)
    p1 = _prop_kernel(y1, src3, dst3)
    xl2, y2 = _mid_call(1.0, True, False)(
        xl1, y1, p1, dinv, be1.reshape(1, D), We2)
    p2 = _prop_kernel(y2, src3, dst3)
    h, xl3, y3 = _mid_call(1.0, False, True)(
        xl2, y2, p2, dinv, be2.reshape(1, D), Wd1)
    p3 = _prop_kernel(y3, src3, dst3)
    xl4, y4 = _mid_call(GAMMA_COEF, True, False)(
        xl3, y3, p3, dinv, bd1.reshape(1, D), Wd2)
    p4 = _prop_kernel(y4, src3, dst3)
    r = _last_call(GAMMA_COEF, False)(xl4, y4, p4, dinv, bd2.reshape(1, D))
    return (h, r)
